# baseline (speedup 1.0000x reference)
import jax
import jax.numpy as jnp
from jax.experimental import pallas as pl


def _row_lse_normalize(x_ref, o_ref):
    v = x_ref[...]
    m = jnp.max(v)
    s = jnp.sum(jnp.exp(v - m))
    o_ref[...] = v - (m + jnp.log(s))


def kernel(x):
    rows, n = x.shape
    sub = 8
    assert n % sub == 0
    cols = n // sub
    x3 = x.reshape(rows * sub, cols)

    out = pl.pallas_call(
        _row_lse_normalize,
        grid=(rows,),
        in_specs=[pl.BlockSpec((sub, cols), lambda i: (i, 0))],
        out_specs=pl.BlockSpec((sub, cols), lambda i: (i, 0)),
        out_shape=jax.ShapeDtypeStruct((rows * sub, cols), x.dtype),
    )(x3)
    return out.reshape(rows, n)
